# baseline (speedup 1.0000x reference)
import functools

import jax
import jax.numpy as jnp
from jax import lax
from jax.experimental import pallas as pl
from jax.experimental.pallas import tpu as pltpu
from jax.experimental.pallas import tpu_sc as plsc

NC = 2
NS = 16
NW = NC * NS
CH = 128
LANES = 16


@jax.jit
def _sc_gather(table_blocks, idx_grid):
  NWg, b_per_w = idx_grid.shape
  n_ch = b_per_w // CH
  D = LANES
  mesh = plsc.VectorSubcoreMesh(
      core_axis_name="c", subcore_axis_name="s", num_cores=NC,
      num_subcores=NS)

  @functools.partial(
      pl.kernel,
      out_type=jax.ShapeDtypeStruct((NWg, b_per_w, D), jnp.float32),
      mesh=mesh,
      scratch_types=[
          pltpu.VMEM((b_per_w,), jnp.int32),
          pltpu.VMEM((b_per_w,), jnp.int32),
          pltpu.VMEM((2, CH, 128), jnp.float32),
          pltpu.VMEM((b_per_w, D), jnp.float32),
          [pltpu.SemaphoreType.DMA] * 4,
      ],
      compiler_params=pltpu.CompilerParams(needs_layout_passes=False),
  )
  def body(table_hbm, idx_hbm, out_hbm, idx_v, bid_v, blocks_v, rows_v,
           sems):
    wid = lax.axis_index("s") * NC + lax.axis_index("c")
    pltpu.sync_copy(idx_hbm.at[wid], idx_v)

    def compute_bids(g, carry):
      o = pl.multiple_of(g * LANES, LANES)
      bid_v[pl.ds(o, LANES)] = idx_v[pl.ds(o, LANES)] >> 3
      return carry

    lax.fori_loop(0, b_per_w // LANES, compute_bids, 0)

    def fire(j):
      return pltpu.async_copy(
          table_hbm.at[bid_v.at[pl.ds(j * CH, CH)]],
          blocks_v.at[j % 2], sems[j % 2])

    copies = [None] * n_ch
    copies[0] = fire(0)
    if n_ch > 1:
      copies[1] = fire(1)

    iota = lax.iota(jnp.int32, LANES)
    for j in range(n_ch):
      copies[j].wait()
      jsplat = jnp.full((LANES,), j % 2, jnp.int32)

      def extract(g, carry, j=j, jsplat=jsplat):
        o = pl.multiple_of(g * LANES, LANES)
        idxc = idx_v[pl.ds(j * CH + o, LANES)]
        colbase = (idxc & 7) << 4
        rloc = iota + o
        grow = rloc + j * CH
        for c in range(D):
          vals = plsc.load_gather(
              blocks_v, [jsplat, rloc, colbase + c])
          plsc.store_scatter(
              rows_v, [grow, jnp.full((LANES,), c, jnp.int32)], vals)
        return carry

      lax.fori_loop(0, CH // LANES, extract, 0)
      if j + 2 < n_ch:
        copies[j + 2] = fire(j + 2)

    pltpu.sync_copy(rows_v, out_hbm.at[wid])

  return body(table_blocks, idx_grid)


def kernel(ordered_embs, nodes):
  V, D = ordered_embs.shape
  B = nodes.shape[0]
  b_per_w = B // NW
  table_blocks = ordered_embs.reshape(V * D // 128, 128)
  idx_grid = nodes.reshape(NW, b_per_w)
  out = _sc_gather(table_blocks, idx_grid)
  return out.reshape(B, D)


# speedup vs baseline: 5.4352x; 5.4352x over previous
import functools

import jax
import jax.numpy as jnp
from jax import lax
from jax.experimental import pallas as pl
from jax.experimental.pallas import tpu as pltpu
from jax.experimental.pallas import tpu_sc as plsc

NC = 2
NS = 16
NW = NC * NS
G = 16
TC = 128


@jax.jit
def _sc_gather_t(tableT, idx):
  D, V = tableT.shape
  (B,) = idx.shape
  b_per_w = B // NW
  n_grp = b_per_w // G
  mesh = plsc.VectorSubcoreMesh(
      core_axis_name="c", subcore_axis_name="s", num_cores=NC,
      num_subcores=NS)

  @functools.partial(
      pl.kernel,
      out_type=jax.ShapeDtypeStruct((D, B), jnp.float32),
      mesh=mesh,
      scratch_types=[
          pltpu.VMEM((b_per_w,), jnp.int32),
          pltpu.VMEM((G, D, TC), jnp.float32),
          pltpu.VMEM((D, b_per_w), jnp.float32),
          [pltpu.SemaphoreType.DMA] * G,
      ],
      compiler_params=pltpu.CompilerParams(needs_layout_passes=False),
  )
  def body(table_hbm, idx_hbm, outT_hbm, idx_v, blocks_v, outblk_v, sems):
    wid = lax.axis_index("s") * NC + lax.axis_index("c")
    base = wid * b_per_w
    pltpu.sync_copy(idx_hbm.at[pl.ds(base, b_per_w)], idx_v)
    iota = lax.iota(jnp.int32, D)

    def fire(c, i):
      a = pl.multiple_of(i & -TC, TC)
      return pltpu.async_copy(
          table_hbm.at[:, pl.ds(a, TC)], blocks_v.at[c], sems[c])

    def drain(c):
      pltpu.make_async_copy(
          table_hbm.at[:, pl.ds(0, TC)], blocks_v.at[c], sems[c]).wait()

    def extract(c, i, pos):
      drain(c)
      col = plsc.load_gather(
          blocks_v,
          [jnp.full((D,), c, jnp.int32), iota,
           jnp.broadcast_to(i & (TC - 1), (D,))])
      plsc.store_scatter(outblk_v, [iota, jnp.broadcast_to(pos, (D,))], col)

    vec0 = idx_v[pl.ds(0, G)]
    for c in range(G):
      fire(c, vec0[c])

    def grp(g, carry):
      o = pl.multiple_of(g * G, G)
      vec = idx_v[pl.ds(o, G)]
      nxt = idx_v[pl.ds(o + G, G)]
      for c in range(G):
        extract(c, vec[c], o + c)
        fire(c, nxt[c])
      return carry

    lax.fori_loop(0, n_grp - 1, grp, 0)

    last = (n_grp - 1) * G
    vecl = idx_v[pl.ds(last, G)]
    for c in range(G):
      extract(c, vecl[c], last + c)

    pltpu.sync_copy(outblk_v, outT_hbm.at[:, pl.ds(base, b_per_w)])

  return body(tableT, idx)


def kernel(ordered_embs, nodes):
  B = nodes.shape[0]
  outT = _sc_gather_t(ordered_embs.T, nodes.reshape(B))
  return outT.T
